# baseline (speedup 1.0000x reference)
import functools

import jax
import jax.numpy as jnp
from jax import lax
from jax.experimental import pallas as pl
from jax.experimental.pallas import tpu as pltpu
from jax.experimental.pallas import tpu_sc as plsc

_DIM = 32
_BATCH = 16384
_FIELDS = 26
_BB = 128


@functools.lru_cache(maxsize=None)
def _build(batch, fields, dim):
    info = plsc.get_sparse_core_info()
    nc, ns = info.num_cores, info.num_subcores
    nw = nc * ns
    n_pairs = fields * (batch // _BB)
    per_w = n_pairs // nw
    nbb = batch // _BB
    assert n_pairs % nw == 0 and per_w % 2 == 0 and per_w >= 6
    pack = 128 // dim

    mesh = plsc.VectorSubcoreMesh(core_axis_name="c", subcore_axis_name="s")

    @functools.partial(
        pl.kernel,
        mesh=mesh,
        out_type=jax.ShapeDtypeStruct((fields, dim, batch), jnp.float32),
        scratch_types=[pltpu.VMEM((_BB,), jnp.int32)] * 2
        + [pltpu.VMEM((_BB,), jnp.int32)] * 2
        + [pltpu.VMEM((_BB, 128), jnp.float32)] * 2
        + [pltpu.VMEM((dim, _BB), jnp.float32)] * 2
        + [pltpu.SemaphoreType.DMA] * 6,
        compiler_params=pltpu.CompilerParams(needs_layout_passes=False),
    )
    def gather_kernel(xt_hbm, table_hbm, out_hbm, *rest):
        ibuf = rest[0:2]
        jbuf = rest[2:4]
        gbuf = rest[4:6]
        obuf = rest[6:8]
        isem = rest[8:10]
        gsem = rest[10:12]
        ssem = rest[12:14]
        wid = lax.axis_index("s") * nc + lax.axis_index("c")
        t0 = wid * per_w

        def fld(t):
            return t // nbb

        def bb(t):
            return (t % nbb) * _BB

        def icopy(t, b):
            return pltpu.make_async_copy(
                xt_hbm.at[fld(t), pl.ds(bb(t), _BB)], ibuf[b], isem[b]
            )

        def gcopy(b):
            return pltpu.make_async_copy(
                table_hbm.at[jbuf[b]], gbuf[b], gsem[b]
            )

        def scopy(t, b):
            return pltpu.make_async_copy(
                obuf[b],
                out_hbm.at[fld(t), :, pl.ds(bb(t), _BB)],
                ssem[b],
            )

        lanes = lax.iota(jnp.int32, 16)

        def jcompute(b):
            for k in range(_BB // 16):
                s = k * 16
                jbuf[b][pl.ds(s, 16)] = ibuf[b][pl.ds(s, 16)] >> 2

        def select(b):
            def body(g, carry):
                s = g * 16
                bvec = s + lanes
                iv = ibuf[b][pl.ds(s, 16)]
                colbase = (iv & (pack - 1)) * dim
                for c in range(dim):
                    v = plsc.load_gather(gbuf[b], [bvec, colbase + c])
                    cfull = jnp.full((16,), c, jnp.int32)
                    plsc.store_scatter(obuf[b], [cfull, bvec], v)
                return carry

            lax.fori_loop(0, _BB // 16, body, 0)

        def step(t, b, with_ssem, more_icopy, more_gather):
            gcopy(b).wait()
            if with_ssem:
                scopy(t0, b).wait()
            select(b)
            scopy(t, b).start()
            if more_icopy:
                icopy(t + 2, b).start()
            if more_gather:
                icopy(t0, 1 - b).wait()
                jcompute(1 - b)
                gcopy(1 - b).start()

        icopy(t0, 0).start()
        icopy(t0 + 1, 1).start()
        icopy(t0, 0).wait()
        jcompute(0)
        gcopy(0).start()
        step(t0, 0, False, True, True)
        step(t0 + 1, 1, False, True, True)

        def group(g, carry):
            t = t0 + 2 * g + 2
            step(t, 0, True, True, True)
            step(t + 1, 1, True, True, True)
            return carry

        lax.fori_loop(0, (per_w - 4) // 2, group, 0)

        step(t0 + per_w - 2, 0, True, False, True)
        step(t0 + per_w - 1, 1, True, False, False)
        scopy(t0, 0).wait()
        scopy(t0, 1).wait()

    return gather_kernel


def kernel(x, weight):
    xt = jnp.swapaxes(x, 0, 1).astype(jnp.int32)
    table = weight.reshape(weight.shape[0] * weight.shape[1] // 128, 128)
    out_t = _build(_BATCH, _FIELDS, _DIM)(xt, table)
    return jnp.transpose(out_t, (2, 0, 1))


# speedup vs baseline: 1.3160x; 1.3160x over previous
import functools

import jax
import jax.numpy as jnp
from jax import lax
from jax.experimental import pallas as pl
from jax.experimental.pallas import tpu as pltpu
from jax.experimental.pallas import tpu_sc as plsc

_DIM = 32
_BATCH = 16384
_FIELDS = 26
_BB = 512
_NBUF = 3


@functools.lru_cache(maxsize=None)
def _build(batch, fields, dim):
    info = plsc.get_sparse_core_info()
    nc, ns = info.num_cores, info.num_subcores
    nw = nc * ns
    nbb = batch // _BB
    n_pairs = fields * nbb
    per_w = n_pairs // nw
    nbuf = _NBUF
    assert n_pairs % nw == 0 and per_w >= nbuf

    mesh = plsc.VectorSubcoreMesh(core_axis_name="c", subcore_axis_name="s")

    @functools.partial(
        pl.kernel,
        mesh=mesh,
        out_type=jax.ShapeDtypeStruct((fields, batch, dim), jnp.float32),
        scratch_types=[pltpu.VMEM((_BB,), jnp.int32)] * per_w
        + [pltpu.VMEM((_BB, dim), jnp.float32)] * nbuf
        + [pltpu.SemaphoreType.DMA] * (1 + 2 * nbuf),
        compiler_params=pltpu.CompilerParams(use_tc_tiling_on_sc=False),
    )
    def gather_kernel(xt_hbm, table_hbm, out_hbm, *rest):
        ibuf = rest[:per_w]
        gbuf = rest[per_w : per_w + nbuf]
        isem = rest[per_w + nbuf]
        gsem = rest[per_w + nbuf + 1 : per_w + 2 * nbuf + 1]
        ssem = rest[per_w + 2 * nbuf + 1 :]
        wid = lax.axis_index("s") * nc + lax.axis_index("c")
        t0 = wid * per_w

        def fld(k):
            return (t0 + k) // nbb

        def bb(k):
            return ((t0 + k) % nbb) * _BB

        def icopy(k):
            return pltpu.make_async_copy(
                xt_hbm.at[fld(k), pl.ds(bb(k), _BB)], ibuf[k], isem
            )

        def gcopy(k, b):
            return pltpu.make_async_copy(
                table_hbm.at[ibuf[k]], gbuf[b], gsem[b]
            )

        def scopy(k, b):
            return pltpu.make_async_copy(
                gbuf[b],
                out_hbm.at[fld(k), pl.ds(bb(k), _BB), :],
                ssem[b],
            )

        for k in range(per_w):
            icopy(k).start()
        for k in range(per_w):
            icopy(k).wait()

        for b in range(nbuf):
            gcopy(b, b).start()
        for k in range(per_w):
            b = k % nbuf
            gcopy(k, b).wait()
            scopy(k, b).start()
            nxt = k + nbuf
            if nxt < per_w:
                scopy(k, b).wait()
                gcopy(nxt, b).start()
        for k in range(per_w - min(nbuf, per_w), per_w):
            scopy(k, k % nbuf).wait()

    return gather_kernel


def kernel(x, weight):
    xt = jnp.swapaxes(x, 0, 1).astype(jnp.int32)
    out_t = _build(_BATCH, _FIELDS, _DIM)(xt, weight)
    return jnp.swapaxes(out_t, 0, 1)


# speedup vs baseline: 1.4802x; 1.1248x over previous
import functools

import jax
import jax.numpy as jnp
from jax import lax
from jax.experimental import pallas as pl
from jax.experimental.pallas import tpu as pltpu
from jax.experimental.pallas import tpu_sc as plsc

_DIM = 32
_BATCH = 16384
_FIELDS = 26
_BB = 512
_NBUF = 3


@functools.lru_cache(maxsize=None)
def _build(batch, fields, dim):
    info = plsc.get_sparse_core_info()
    nc, ns = info.num_cores, info.num_subcores
    nw = nc * ns
    nbb = batch // _BB
    n_pairs = fields * nbb
    per_w = n_pairs // nw
    nbuf = _NBUF
    assert n_pairs % nw == 0 and per_w >= nbuf

    mesh = plsc.VectorSubcoreMesh(core_axis_name="c", subcore_axis_name="s")

    @functools.partial(
        pl.kernel,
        mesh=mesh,
        out_type=jax.ShapeDtypeStruct((fields, batch, dim), jnp.float32),
        scratch_types=[pltpu.VMEM((_BB,), jnp.int32)] * per_w
        + [pltpu.VMEM((_BB, dim), jnp.float32)] * nbuf
        + [pltpu.SemaphoreType.DMA] * (1 + 2 * nbuf),
        compiler_params=pltpu.CompilerParams(use_tc_tiling_on_sc=False),
    )
    def gather_kernel(xt_hbm, table_hbm, out_hbm, *rest):
        ibuf = rest[:per_w]
        gbuf = rest[per_w : per_w + nbuf]
        isem = rest[per_w + nbuf]
        gsem = rest[per_w + nbuf + 1 : per_w + 2 * nbuf + 1]
        ssem = rest[per_w + 2 * nbuf + 1 :]
        wid = lax.axis_index("s") * nc + lax.axis_index("c")
        t0 = wid * per_w

        def fld(k):
            return (t0 + k) // nbb

        def bb(k):
            return ((t0 + k) % nbb) * _BB

        def icopy(k):
            return pltpu.make_async_copy(
                xt_hbm.at[fld(k), pl.ds(bb(k), _BB)], ibuf[k], isem
            )

        def gcopy(k, b):
            return pltpu.make_async_copy(
                table_hbm.at[ibuf[k]], gbuf[b], gsem[b]
            )

        def scopy(k, b):
            return pltpu.make_async_copy(
                gbuf[b],
                out_hbm.at[fld(k), pl.ds(bb(k), _BB), :],
                ssem[b],
            )

        for k in range(per_w):
            icopy(k).start()
        for k in range(per_w):
            icopy(k).wait()

        for b in range(nbuf):
            gcopy(b, b).start()
        for k in range(per_w):
            b = k % nbuf
            gcopy(k, b).wait()
            scopy(k, b).start()
            nxt = k + nbuf
            if nxt < per_w:
                scopy(k, b).wait()
                gcopy(nxt, b).start()
        for k in range(per_w - min(nbuf, per_w), per_w):
            scopy(k, k % nbuf).wait()

    return gather_kernel


@functools.lru_cache(maxsize=None)
def _build_k1(vocab, dim):
    info = plsc.get_sparse_core_info()
    nc, ns = info.num_cores, info.num_subcores
    nw = nc * ns
    rows = vocab * dim // 128
    main_cols = (vocab // 128) * 128 - ((vocab // 128) % nw) * 128
    n_blocks = main_cols // 128
    per_w = n_blocks // nw
    n_extra = vocab // 128 - n_blocks
    tail = vocab - (vocab // 128) * 128
    assert per_w % 2 == 0 and per_w >= 6

    mesh = plsc.VectorSubcoreMesh(core_axis_name="c", subcore_axis_name="s")

    @functools.partial(
        pl.kernel,
        mesh=mesh,
        out_type=jax.ShapeDtypeStruct((rows, 128), jnp.float32),
        scratch_types=[pltpu.VMEM((dim, 128), jnp.float32)] * 2
        + [pltpu.VMEM((dim, 128), jnp.float32)] * 2
        + [pltpu.VMEM((dim, tail), jnp.float32), pltpu.VMEM((tail * dim // 128, 128), jnp.float32)]
        + [pltpu.SemaphoreType.DMA] * 4,
        compiler_params=pltpu.CompilerParams(needs_layout_passes=False),
    )
    def t_kernel(wt_hbm, out_hbm, *rest):
        ibuf = rest[0:2]
        obuf = rest[2:4]
        ibuf_t, obuf_t = rest[4], rest[5]
        isem = rest[6:8]
        osem = rest[8:10]
        wid = lax.axis_index("s") * nc + lax.axis_index("c")
        lanes = lax.iota(jnp.int32, 16)

        def c0(m):
            return pl.multiple_of((wid * per_w + m) * 128, 128)

        def icopy(m, b):
            return pltpu.make_async_copy(
                wt_hbm.at[:, pl.ds(c0(m), 128)], ibuf[b], isem[b]
            )

        def ocopy(m, b):
            return pltpu.make_async_copy(
                obuf[b],
                out_hbm.at[pl.ds(pl.multiple_of(c0(m) // 4, 32), dim), :],
                osem[b],
            )

        def transpose(src, dst, njr):
            @plsc.parallel_loop(0, njr, unroll=2)
            def body(jr):
                for t in range(8):
                    cvec = (t % 2) * 16 + lanes
                    vloc = jnp.full((16,), 0, jnp.int32) + (jr * 4 + t // 2)
                    v = plsc.load_gather(src, [cvec, vloc])
                    dst[jr, pl.ds(16 * t, 16)] = v

        def step(m, b, with_osem, more_icopy):
            icopy(0, b).wait()
            if with_osem:
                ocopy(0, b).wait()
            transpose(ibuf[b], obuf[b], dim)
            ocopy(m, b).start()
            if more_icopy:
                icopy(m + 2, b).start()

        icopy(0, 0).start()
        icopy(1, 1).start()
        step(0, 0, False, True)
        step(1, 1, False, True)

        def group(g, carry):
            m = 2 * g + 2
            step(m, 0, True, True)
            step(m + 1, 1, True, True)
            return carry

        lax.fori_loop(0, (per_w - 4) // 2, group, 0)

        step(per_w - 2, 0, True, False)
        step(per_w - 1, 1, True, False)
        ocopy(0, 0).wait()
        ocopy(0, 1).wait()

        @pl.when(wid < n_extra)
        def _():
            ce = pl.multiple_of((n_blocks + wid) * 128, 128)
            cp = pltpu.make_async_copy(
                wt_hbm.at[:, pl.ds(ce, 128)], ibuf[0], isem[0]
            )
            cp.start()
            cp.wait()
            transpose(ibuf[0], obuf[0], dim)
            cp2 = pltpu.make_async_copy(
                obuf[0],
                out_hbm.at[pl.ds(pl.multiple_of(ce // 4, 32), dim), :],
                osem[0],
            )
            cp2.start()
            cp2.wait()

        @pl.when(wid == n_extra)
        def _():
            ct = (n_blocks + n_extra) * 128
            cp = pltpu.make_async_copy(
                wt_hbm.at[:, pl.ds(ct, tail)], ibuf_t, isem[0]
            )
            cp.start()
            cp.wait()
            transpose(ibuf_t, obuf_t, tail * dim // 128)
            cp2 = pltpu.make_async_copy(
                obuf_t,
                out_hbm.at[pl.ds(ct // 4, tail * dim // 128), :],
                osem[0],
            )
            cp2.start()
            cp2.wait()

    return t_kernel


def kernel(x, weight):
    xt = jnp.swapaxes(x, 0, 1).astype(jnp.int32)
    table = _build_k1(weight.shape[0], weight.shape[1])(weight.T)
    table = table.reshape(weight.shape)
    out_t = _build(_BATCH, _FIELDS, _DIM)(xt, table)
    return jnp.swapaxes(out_t, 0, 1)
